# baseline (speedup 1.0000x reference)
import jax
import jax.numpy as jnp
from jax.experimental import pallas as pl

N = 10000
E = 320000
H = 8
D = 16
EMB = 128
NQ = 8000
B = 1024
L = 200


def _gat(x, src, dst, W, al, ar, b):
    feat = (x @ W).reshape(N, H, D)
    el = jnp.sum(feat * al[None, :, :], axis=-1)
    er = jnp.sum(feat * ar[None, :, :], axis=-1)
    e = jax.nn.leaky_relu(el[src] + er[dst], negative_slope=0.2)
    emax = jax.ops.segment_max(e, dst, num_segments=N)
    ee = jnp.exp(e - emax[dst])
    den = jax.ops.segment_sum(ee, dst, num_segments=N)
    a = ee / den[dst]
    msg = feat[src] * a[:, :, None]
    rst = jax.ops.segment_sum(msg, dst, num_segments=N)
    rst = rst + x.reshape(N, H, D) + b.reshape(1, H, D)
    rst = jax.nn.elu(rst)
    return rst.reshape(N, H * D)


def _copy_kernel(x_ref, o_ref):
    o_ref[...] = x_ref[...]


def kernel(x1, x2, edge_index1, edge_index2, pad_ques, W1, al1, ar1, b1, W2, al2, ar2, b2, attnVec):
    emb1 = _gat(x1, edge_index1[0], edge_index1[1], W1, al1, ar1, b1)
    emb2 = _gat(x2, edge_index2[0], edge_index2[1], W2, al2, ar2, b2)
    ques = jnp.stack([emb1[:NQ], emb2[:NQ]], axis=1)
    path_weight = jax.nn.softmax(jnp.matmul(ques, attnVec), axis=1)
    ques_emb = jnp.sum(ques * path_weight, axis=1)
    ques_emb = pl.pallas_call(
        _copy_kernel,
        out_shape=jax.ShapeDtypeStruct((NQ, EMB), jnp.float32),
    )(ques_emb)
    batch_ques_emb = jnp.take(ques_emb, pad_ques, axis=0)
    return batch_ques_emb


# speedup vs baseline: 26.0259x; 26.0259x over previous
import functools

import jax
import jax.numpy as jnp
from jax import lax
from jax.experimental import pallas as pl
from jax.experimental.pallas import tpu as pltpu
from jax.experimental.pallas import tpu_sc as plsc

N = 10000
E = 320000
H = 8
D = 16
EMB = 128
NQ = 8000
B = 1024
L = 200
P = 2

NTILES = 16
CHUNK = 64
EW = 64
IDXW = 128
EPT = 20480
E_PAD = EPT * NTILES
N_ACC = 10112
ZROWS = N_ACC // NTILES
NQF = 8192
QPT = NQF // NTILES
QCH = 32
TOK = B * L
NW = 32
TPW = TOK // NW
LCH = 640


def _prep_body(x_ref, w_ref, ael_ref, aer_ref, feat_ref, el_ref, er_ref):
    hi = jax.lax.Precision.HIGHEST
    f = jnp.dot(x_ref[0], w_ref[0], precision=hi,
                preferred_element_type=jnp.float32)
    feat_ref[0] = f
    el_ref[0] = jnp.dot(f, ael_ref[0], precision=hi,
                        preferred_element_type=jnp.float32)
    er_ref[0] = jnp.dot(f, aer_ref[0], precision=hi,
                        preferred_element_type=jnp.float32)


def _prep(x_stack, w_stack, ael, aer):
    rb = 2000
    grid = (P, N // rb)
    return pl.pallas_call(
        _prep_body,
        grid=grid,
        in_specs=[
            pl.BlockSpec((1, rb, EMB), lambda p, i: (p, i, 0)),
            pl.BlockSpec((1, EMB, EMB), lambda p, i: (p, 0, 0)),
            pl.BlockSpec((1, EMB, EMB), lambda p, i: (p, 0, 0)),
            pl.BlockSpec((1, EMB, EMB), lambda p, i: (p, 0, 0)),
        ],
        out_specs=[
            pl.BlockSpec((1, rb, EMB), lambda p, i: (p, i, 0)),
            pl.BlockSpec((1, rb, EMB), lambda p, i: (p, i, 0)),
            pl.BlockSpec((1, rb, EMB), lambda p, i: (p, i, 0)),
        ],
        out_shape=[
            jax.ShapeDtypeStruct((P, N, EMB), jnp.float32),
            jax.ShapeDtypeStruct((P, N, EMB), jnp.float32),
            jax.ShapeDtypeStruct((P, N, EMB), jnp.float32),
        ],
    )(x_stack, w_stack, ael, aer)


def _den_body(el_hbm, er_hbm, srcb_hbm, dstb_hbm, dstu_hbm, zacc_hbm,
              den_out, den_sh, i_src, i_dstb, i_dstu, sbuf, elbuf, eeb,
              sem_er, sem_el):
    c = lax.axis_index("c")
    s = lax.axis_index("s")

    pltpu.sync_copy(zacc_hbm, den_sh.at[pl.ds(s * ZROWS, ZROWS)])

    def z_body(k, _):
        for t in range(8):
            eeb[k, pl.ds(16 * t, 16)] = jnp.zeros((16,), jnp.float32)
        return ()
    lax.fori_loop(0, CHUNK, z_body, ())
    plsc.subcore_barrier()

    rows_per_tile = EPT // EW
    tile_row0 = s * rows_per_tile

    def chunk_body(jj, _):
        row0 = tile_row0 + 16 * jj
        pltpu.sync_copy(srcb_hbm.at[c, pl.ds(row0, 16)], i_src)
        pltpu.sync_copy(dstb_hbm.at[c, pl.ds(row0, 16)], i_dstb)
        pltpu.sync_copy(dstu_hbm.at[c, pl.ds(row0, 16)], i_dstu)
        for g in range(16):
            h_er = pltpu.async_copy(er_hbm.at[i_dstb.at[g]], sbuf, sem_er)
            h_el = pltpu.async_copy(el_hbm.at[i_src.at[g]], elbuf, sem_el)
            h_er.wait()
            h_el.wait()

            def ee_body(k, _):
                sv = sbuf[k, pl.ds(0, 16)] + elbuf[k, pl.ds(0, 16)]
                eeb[k, pl.ds(0, 16)] = jnp.exp(jnp.maximum(sv, 0.2 * sv))
                return ()
            lax.fori_loop(0, CHUNK, ee_body, ())

            pltpu.sync_copy(eeb, den_sh.at[i_dstu.at[g]], add=True)
        return ()

    lax.fori_loop(0, EPT // (16 * CHUNK), chunk_body, ())
    plsc.subcore_barrier()

    r0 = s * QPT
    pltpu.sync_copy(den_sh.at[pl.ds(r0, QPT)], den_out.at[c, pl.ds(r0, QPT)])


def _den_kernel(el16, er16, srcb, dstb, dstu, zacc):
    mesh = plsc.VectorSubcoreMesh(core_axis_name="c", subcore_axis_name="s",
                                  num_cores=2, num_subcores=16)
    f = pl.kernel(
        _den_body,
        out_type=jax.ShapeDtypeStruct((P, NQF, EMB), jnp.float32),
        mesh=mesh,
        scratch_types=[
            pltpu.VMEM_SHARED((N_ACC, EMB), jnp.float32),
            pltpu.VMEM((16, EW), jnp.int32),
            pltpu.VMEM((16, EW), jnp.int32),
            pltpu.VMEM((16, EW), jnp.int32),
            pltpu.VMEM((CHUNK, EMB), jnp.float32),
            pltpu.VMEM((CHUNK, EMB), jnp.float32),
            pltpu.VMEM((CHUNK, EMB), jnp.float32),
            pltpu.SemaphoreType.DMA,
            pltpu.SemaphoreType.DMA,
        ],
    )
    return f(el16, er16, srcb, dstb, dstu, zacc)


def _acc_body(feat_hbm, el_hbm, er_hbm, srcb_hbm, dstb_hbm, dstu_hbm,
              den_hbm, x_hbm, b_hbm, zacc_hbm, out_hbm,
              acc_sh, i_src, i_dstb, i_dstu, sbuf, elbuf, featb, bb,
              sem_f, sem_er, sem_el):
    c = lax.axis_index("c")
    s = lax.axis_index("s")

    pltpu.sync_copy(zacc_hbm, acc_sh.at[pl.ds(s * ZROWS, ZROWS)])
    plsc.subcore_barrier()

    rows_per_tile = EPT // EW
    tile_row0 = s * rows_per_tile

    def chunk_body(jj, _):
        row0 = tile_row0 + 16 * jj
        pltpu.sync_copy(srcb_hbm.at[c, pl.ds(row0, 16)], i_src)
        pltpu.sync_copy(dstb_hbm.at[c, pl.ds(row0, 16)], i_dstb)
        pltpu.sync_copy(dstu_hbm.at[c, pl.ds(row0, 16)], i_dstu)
        for g in range(16):
            h_f = pltpu.async_copy(feat_hbm.at[i_src.at[g]], featb, sem_f)
            h_er = pltpu.async_copy(er_hbm.at[i_dstb.at[g]], sbuf, sem_er)
            h_el = pltpu.async_copy(el_hbm.at[i_src.at[g]], elbuf, sem_el)
            h_er.wait()
            h_el.wait()
            h_f.wait()

            def mul_body(k, _):
                sv = sbuf[k, pl.ds(0, 16)] + elbuf[k, pl.ds(0, 16)]
                eev = jnp.exp(jnp.maximum(sv, 0.2 * sv))
                for h in range(H):
                    featb[k, pl.ds(16 * h, 16)] = (
                        featb[k, pl.ds(16 * h, 16)] * eev[h])
                return ()
            lax.fori_loop(0, CHUNK, mul_body, ())

            pltpu.sync_copy(featb, acc_sh.at[i_dstu.at[g]], add=True)
        return ()

    lax.fori_loop(0, EPT // (16 * CHUNK), chunk_body, ())
    plsc.subcore_barrier()

    pltpu.sync_copy(b_hbm.at[c], bb)

    def fin_chunk(j, _):
        r0 = s * QPT + j * QCH
        pltpu.sync_copy(acc_sh.at[pl.ds(r0, QCH)], featb.at[pl.ds(0, QCH)])
        pltpu.sync_copy(den_hbm.at[c, pl.ds(r0, QCH)],
                        elbuf.at[pl.ds(0, QCH)])
        pltpu.sync_copy(x_hbm.at[pl.ds(c * N + r0, QCH)],
                        sbuf.at[pl.ds(0, QCH)])

        def fin_row(r, _):
            dvv = jnp.maximum(elbuf[r, pl.ds(0, 16)], 1e-30)
            for h in range(H):
                num = featb[r, pl.ds(16 * h, 16)]
                dv = dvv[h]
                rst = (num / dv + sbuf[r, pl.ds(16 * h, 16)]
                       + bb[pl.ds(16 * h, 16)])
                featb[r, pl.ds(16 * h, 16)] = jnp.where(
                    rst > 0.0, rst, jnp.exp(rst) - 1.0)
            return ()
        lax.fori_loop(0, QCH, fin_row, ())
        pltpu.sync_copy(featb.at[pl.ds(0, QCH)], out_hbm.at[c, pl.ds(r0, QCH)])
        return ()

    lax.fori_loop(0, QPT // QCH, fin_chunk, ())


def _acc_kernel(feat, el16, er16, srcb, dstb, dstu, den, x_stack, b_stack,
                zacc):
    mesh = plsc.VectorSubcoreMesh(core_axis_name="c", subcore_axis_name="s",
                                  num_cores=2, num_subcores=16)
    f = pl.kernel(
        _acc_body,
        out_type=jax.ShapeDtypeStruct((P, NQF, EMB), jnp.float32),
        mesh=mesh,
        scratch_types=[
            pltpu.VMEM_SHARED((N_ACC, EMB), jnp.float32),
            pltpu.VMEM((16, EW), jnp.int32),
            pltpu.VMEM((16, EW), jnp.int32),
            pltpu.VMEM((16, EW), jnp.int32),
            pltpu.VMEM((CHUNK, EMB), jnp.float32),
            pltpu.VMEM((CHUNK, EMB), jnp.float32),
            pltpu.VMEM((CHUNK, EMB), jnp.float32),
            pltpu.VMEM((EMB,), jnp.float32),
            pltpu.SemaphoreType.DMA,
            pltpu.SemaphoreType.DMA,
            pltpu.SemaphoreType.DMA,
        ],
    )
    return f(feat, el16, er16, srcb, dstb, dstu, den, x_stack, b_stack, zacc)


def _fuse_body(emb_ref, av_ref, out_ref):
    e = emb_ref[...]
    av = av_ref[...]
    logits = jnp.sum(e * av[:, None, :], axis=-1)
    m = jnp.max(logits, axis=0, keepdims=True)
    w = jnp.exp(logits - m)
    w = w / jnp.sum(w, axis=0, keepdims=True)
    out_ref[...] = e[0] * w[0][:, None] + e[1] * w[1][:, None]


def _fuse(emb, av):
    return pl.pallas_call(
        _fuse_body,
        out_shape=jax.ShapeDtypeStruct((NQF, EMB), jnp.float32),
    )(emb, av)


def _lookup_body(tab_hbm, idx_hbm, out_hbm, idxb, rowsb, sem):
    c = lax.axis_index("c")
    s = lax.axis_index("s")
    wid = s * 2 + c
    pltpu.sync_copy(idx_hbm.at[wid], idxb)

    def chunk(j, _):
        hs = []
        for g in range(LCH // IDXW):
            hs.append(pltpu.async_copy(
                tab_hbm.at[idxb.at[5 * j + g]],
                rowsb.at[pl.ds(IDXW * g, IDXW)], sem))
        for h in hs:
            h.wait()
        pltpu.sync_copy(rowsb, out_hbm.at[pl.ds(wid * TPW + j * LCH, LCH)])
        return ()

    lax.fori_loop(0, TPW // LCH, chunk, ())


def _lookup(tab, idx2d):
    mesh = plsc.VectorSubcoreMesh(core_axis_name="c", subcore_axis_name="s", num_cores=2, num_subcores=16)
    f = pl.kernel(
        _lookup_body,
        out_type=jax.ShapeDtypeStruct((TOK, EMB), jnp.float32),
        mesh=mesh,
        scratch_types=[
            pltpu.VMEM((TPW // IDXW, IDXW), jnp.int32),
            pltpu.VMEM((LCH, EMB), jnp.float32),
            pltpu.SemaphoreType.DMA,
        ],
    )
    return f(tab, idx2d)


def _blockdiag128(a):
    m = (a[:, :, None] * jnp.eye(H, dtype=a.dtype)[:, None, :]).reshape(EMB, H)
    return jnp.pad(m, ((0, 0), (0, EMB - H)))


def kernel(x1, x2, edge_index1, edge_index2, pad_ques, W1, al1, ar1, b1,
           W2, al2, ar2, b2, attnVec):
    x_stack3 = jnp.stack([x1, x2])
    w_stack = jnp.stack([W1, W2])
    ael = jnp.stack([_blockdiag128(al1), _blockdiag128(al2)])
    aer = jnp.stack([_blockdiag128(ar1), _blockdiag128(ar2)])

    feat3, el3, er3 = _prep(x_stack3, w_stack, ael, aer)
    feat = feat3.reshape(P * N, EMB)
    el16 = el3.reshape(P * N, EMB)
    er16 = er3.reshape(P * N, EMB)

    pad_e = E_PAD - E
    src = jnp.stack([edge_index1[0], edge_index2[0] + N])
    dst = jnp.stack([edge_index1[1], edge_index2[1]])
    srcb = jnp.pad(src, ((0, 0), (0, pad_e))).reshape(P, E_PAD // EW, EW)
    dstb = jnp.pad(jnp.stack([edge_index1[1], edge_index2[1] + N]),
                   ((0, 0), (0, pad_e))).reshape(P, E_PAD // EW, EW)
    dstu = jnp.pad(dst, ((0, 0), (0, pad_e)),
                   constant_values=N).reshape(P, E_PAD // EW, EW)

    x_stack = jnp.concatenate([x1, x2])
    b_stack = jnp.stack([b1, b2])
    zacc = jnp.zeros((ZROWS, EMB), jnp.float32)

    den = _den_kernel(el16, er16, srcb, dstb, dstu, zacc)
    emb = _acc_kernel(feat, el16, er16, srcb, dstb, dstu, den, x_stack,
                      b_stack, zacc)

    ques_emb = _fuse(emb, attnVec.reshape(1, EMB))

    idx3d = pad_ques.reshape(NW, TPW // IDXW, IDXW)
    out = _lookup(ques_emb, idx3d)
    return out.reshape(B, L, EMB)


# speedup vs baseline: 38.9415x; 1.4963x over previous
import functools

import jax
import jax.numpy as jnp
from jax import lax
from jax.experimental import pallas as pl
from jax.experimental.pallas import tpu as pltpu
from jax.experimental.pallas import tpu_sc as plsc

N = 10000
E = 320000
H = 8
D = 16
EMB = 128
NQ = 8000
B = 1024
L = 200
P = 2

NTILES = 16
CHUNK = 48
NSUB = 8
NSUPER = 53
EPT = CHUNK * NSUB * NSUPER
E_PAD = EPT * NTILES
NT = 2 * N + 8
N_ACC = 10112
ZROWS = N_ACC // NTILES
NQF = 8192
QPT = NQF // NTILES
QCH = 32
DUM = N
IDXW = 128
TOK = B * L
NW = 32
TPW = TOK // NW
LCH = 640


def _prep_body(x_ref, w_ref, ael_ref, aer_ref, feat_ref, el_ref, er_ref):
    hi = jax.lax.Precision.HIGHEST
    f = jnp.dot(x_ref[0], w_ref[0], precision=hi,
                preferred_element_type=jnp.float32)
    feat_ref[0] = f
    el_ref[0] = jnp.dot(f, ael_ref[0], precision=hi,
                        preferred_element_type=jnp.float32)
    er_ref[0] = jnp.dot(f, aer_ref[0], precision=hi,
                        preferred_element_type=jnp.float32)


def _prep(x_stack, w_stack, ael, aer):
    rb = 2000
    grid = (P, N // rb)
    return pl.pallas_call(
        _prep_body,
        grid=grid,
        in_specs=[
            pl.BlockSpec((1, rb, EMB), lambda p, i: (p, i, 0)),
            pl.BlockSpec((1, EMB, EMB), lambda p, i: (p, 0, 0)),
            pl.BlockSpec((1, EMB, EMB), lambda p, i: (p, 0, 0)),
            pl.BlockSpec((1, EMB, EMB), lambda p, i: (p, 0, 0)),
        ],
        out_specs=[
            pl.BlockSpec((1, rb, EMB), lambda p, i: (p, i, 0)),
            pl.BlockSpec((1, rb, EMB), lambda p, i: (p, i, 0)),
            pl.BlockSpec((1, rb, EMB), lambda p, i: (p, i, 0)),
        ],
        out_shape=[
            jax.ShapeDtypeStruct((P, N, EMB), jnp.float32),
            jax.ShapeDtypeStruct((P, N, EMB), jnp.float32),
            jax.ShapeDtypeStruct((P, N, EMB), jnp.float32),
        ],
    )(x_stack, w_stack, ael, aer)


def _edge_pipeline(c, s, srcb_hbm, dstb_hbm, i_src1, i_dstb1, i_dus,
                   fire_fn, compute_fn, scatter_fn, sems_s):
    cN = c * N
    tile0 = s * EPT

    def conv(t8, which):
        for t in range(CHUNK // 16):
            i_dus[which][pl.ds(16 * t, 16)] = (
                i_dstb1[pl.ds(CHUNK * t8 + 16 * t, 16)] - cN)

    def super_body(sc, _):
        base = tile0 + sc * (NSUB * CHUNK)
        pltpu.sync_copy(srcb_hbm.at[c, pl.ds(base, NSUB * CHUNK)], i_src1)
        pltpu.sync_copy(dstb_hbm.at[c, pl.ds(base, NSUB * CHUNK)], i_dstb1)
        hs = fire_fn(0, 0)
        sc_h = [None, None]
        for t8 in range(NSUB):
            bset = t8 & 1
            oth = bset ^ 1
            hs_next = None
            if t8 < NSUB - 1:
                if sc_h[oth] is not None:
                    sc_h[oth].wait()
                    sc_h[oth] = None
                hs_next = fire_fn(t8 + 1, oth)
            for h in hs:
                h.wait()
            compute_fn(bset)
            conv(t8, bset)
            sc_h[bset] = scatter_fn(bset, i_dus[bset], sems_s[bset])
            hs = hs_next
        for x in sc_h:
            if x is not None:
                x.wait()
        return ()

    lax.fori_loop(0, NSUPER, super_body, ())


def _den_body(el_hbm, er_hbm, srcb_hbm, dstb_hbm, zacc_hbm, den_out,
              den_sh, sbuf0, sbuf1, elbuf0, elbuf1, eeb0, eeb1,
              i_src1, i_dstb1, i_du0, i_du1,
              sem_g0, sem_g1, sem_s0, sem_s1):
    c = lax.axis_index("c")
    s = lax.axis_index("s")
    sbufs = [sbuf0, sbuf1]
    elbufs = [elbuf0, elbuf1]
    eebs = [eeb0, eeb1]
    sems_g = [sem_g0, sem_g1]

    pltpu.sync_copy(zacc_hbm, den_sh.at[pl.ds(s * ZROWS, ZROWS)])

    def z_body(k, _):
        for t in range(8):
            eeb0[k, pl.ds(16 * t, 16)] = jnp.zeros((16,), jnp.float32)
            eeb1[k, pl.ds(16 * t, 16)] = jnp.zeros((16,), jnp.float32)
        return ()
    lax.fori_loop(0, CHUNK, z_body, ())
    plsc.subcore_barrier()

    def fire(t8, bset):
        idx_s = i_src1.at[pl.ds(CHUNK * t8, CHUNK)]
        idx_d = i_dstb1.at[pl.ds(CHUNK * t8, CHUNK)]
        return [
            pltpu.async_copy(er_hbm.at[idx_d], sbufs[bset], sems_g[bset]),
            pltpu.async_copy(el_hbm.at[idx_s], elbufs[bset], sems_g[bset]),
        ]

    def compute(bset):
        sb, eb, ee = sbufs[bset], elbufs[bset], eebs[bset]

        def ee_body(k, _):
            sv = sb[k, pl.ds(0, 16)] + eb[k, pl.ds(0, 16)]
            ee[k, pl.ds(0, 16)] = jnp.exp(jnp.maximum(sv, 0.2 * sv))
            return ()
        lax.fori_loop(0, CHUNK, ee_body, ())

    def scatter(bset, i_du, sem):
        return pltpu.async_copy(eebs[bset], den_sh.at[i_du], sem, add=True)

    _edge_pipeline(c, s, srcb_hbm, dstb_hbm, i_src1, i_dstb1,
                   [i_du0, i_du1], fire, compute, scatter, [sem_s0, sem_s1])
    plsc.subcore_barrier()

    r0 = s * QPT
    pltpu.sync_copy(den_sh.at[pl.ds(r0, QPT)], den_out.at[c, pl.ds(r0, QPT)])


def _den_kernel(el16, er16, srcb, dstb, zacc):
    mesh = plsc.VectorSubcoreMesh(core_axis_name="c", subcore_axis_name="s",
                                  num_cores=2, num_subcores=16)
    f = pl.kernel(
        _den_body,
        out_type=jax.ShapeDtypeStruct((P, NQF, EMB), jnp.float32),
        mesh=mesh,
        scratch_types=[
            pltpu.VMEM_SHARED((N_ACC, EMB), jnp.float32),
            pltpu.VMEM((CHUNK, EMB), jnp.float32),
            pltpu.VMEM((CHUNK, EMB), jnp.float32),
            pltpu.VMEM((CHUNK, EMB), jnp.float32),
            pltpu.VMEM((CHUNK, EMB), jnp.float32),
            pltpu.VMEM((CHUNK, EMB), jnp.float32),
            pltpu.VMEM((CHUNK, EMB), jnp.float32),
            pltpu.VMEM((NSUB * CHUNK,), jnp.int32),
            pltpu.VMEM((NSUB * CHUNK,), jnp.int32),
            pltpu.VMEM((CHUNK,), jnp.int32),
            pltpu.VMEM((CHUNK,), jnp.int32),
            pltpu.SemaphoreType.DMA,
            pltpu.SemaphoreType.DMA,
            pltpu.SemaphoreType.DMA,
            pltpu.SemaphoreType.DMA,
        ],
    )
    return f(el16, er16, srcb, dstb, zacc)


def _acc_body(feat_hbm, el_hbm, er_hbm, srcb_hbm, dstb_hbm, den_hbm,
              x_hbm, b_hbm, zacc_hbm, out_hbm,
              acc_sh, sbuf0, sbuf1, elbuf0, elbuf1, featb0, featb1,
              i_src1, i_dstb1, i_du0, i_du1, bb,
              sem_g0, sem_g1, sem_s0, sem_s1):
    c = lax.axis_index("c")
    s = lax.axis_index("s")
    sbufs = [sbuf0, sbuf1]
    elbufs = [elbuf0, elbuf1]
    featbs = [featb0, featb1]
    sems_g = [sem_g0, sem_g1]

    pltpu.sync_copy(zacc_hbm, acc_sh.at[pl.ds(s * ZROWS, ZROWS)])
    plsc.subcore_barrier()

    def fire(t8, bset):
        idx_s = i_src1.at[pl.ds(CHUNK * t8, CHUNK)]
        idx_d = i_dstb1.at[pl.ds(CHUNK * t8, CHUNK)]
        return [
            pltpu.async_copy(feat_hbm.at[idx_s], featbs[bset], sems_g[bset]),
            pltpu.async_copy(er_hbm.at[idx_d], sbufs[bset], sems_g[bset]),
            pltpu.async_copy(el_hbm.at[idx_s], elbufs[bset], sems_g[bset]),
        ]

    def compute(bset):
        sb, eb, fb = sbufs[bset], elbufs[bset], featbs[bset]

        def mul_body(k, _):
            sv = sb[k, pl.ds(0, 16)] + eb[k, pl.ds(0, 16)]
            eev = jnp.exp(jnp.maximum(sv, 0.2 * sv))
            for h in range(H):
                fb[k, pl.ds(16 * h, 16)] = fb[k, pl.ds(16 * h, 16)] * eev[h]
            return ()
        lax.fori_loop(0, CHUNK, mul_body, ())

    def scatter(bset, i_du, sem):
        return pltpu.async_copy(featbs[bset], acc_sh.at[i_du], sem, add=True)

    _edge_pipeline(c, s, srcb_hbm, dstb_hbm, i_src1, i_dstb1,
                   [i_du0, i_du1], fire, compute, scatter, [sem_s0, sem_s1])
    plsc.subcore_barrier()

    pltpu.sync_copy(b_hbm.at[c], bb)

    def fin_chunk(j, _):
        r0 = s * QPT + j * QCH
        pltpu.sync_copy(acc_sh.at[pl.ds(r0, QCH)], featb0.at[pl.ds(0, QCH)])
        pltpu.sync_copy(den_hbm.at[c, pl.ds(r0, QCH)],
                        elbuf0.at[pl.ds(0, QCH)])
        pltpu.sync_copy(x_hbm.at[pl.ds(c * N + r0, QCH)],
                        sbuf0.at[pl.ds(0, QCH)])

        def fin_row(r, _):
            dvv = jnp.maximum(elbuf0[r, pl.ds(0, 16)], 1e-30)
            for h in range(H):
                num = featb0[r, pl.ds(16 * h, 16)]
                dv = dvv[h]
                rst = (num / dv + sbuf0[r, pl.ds(16 * h, 16)]
                       + bb[pl.ds(16 * h, 16)])
                featb0[r, pl.ds(16 * h, 16)] = jnp.where(
                    rst > 0.0, rst, jnp.exp(rst) - 1.0)
            return ()
        lax.fori_loop(0, QCH, fin_row, ())
        pltpu.sync_copy(featb0.at[pl.ds(0, QCH)],
                        out_hbm.at[c, pl.ds(r0, QCH)])
        return ()

    lax.fori_loop(0, QPT // QCH, fin_chunk, ())


def _acc_kernel(feat, el16, er16, srcb, dstb, den, x_stack, b_stack, zacc):
    mesh = plsc.VectorSubcoreMesh(core_axis_name="c", subcore_axis_name="s",
                                  num_cores=2, num_subcores=16)
    f = pl.kernel(
        _acc_body,
        out_type=jax.ShapeDtypeStruct((P, NQF, EMB), jnp.float32),
        mesh=mesh,
        scratch_types=[
            pltpu.VMEM_SHARED((N_ACC, EMB), jnp.float32),
            pltpu.VMEM((CHUNK, EMB), jnp.float32),
            pltpu.VMEM((CHUNK, EMB), jnp.float32),
            pltpu.VMEM((CHUNK, EMB), jnp.float32),
            pltpu.VMEM((CHUNK, EMB), jnp.float32),
            pltpu.VMEM((CHUNK, EMB), jnp.float32),
            pltpu.VMEM((CHUNK, EMB), jnp.float32),
            pltpu.VMEM((NSUB * CHUNK,), jnp.int32),
            pltpu.VMEM((NSUB * CHUNK,), jnp.int32),
            pltpu.VMEM((CHUNK,), jnp.int32),
            pltpu.VMEM((CHUNK,), jnp.int32),
            pltpu.VMEM((EMB,), jnp.float32),
            pltpu.SemaphoreType.DMA,
            pltpu.SemaphoreType.DMA,
            pltpu.SemaphoreType.DMA,
            pltpu.SemaphoreType.DMA,
        ],
    )
    return f(feat, el16, er16, srcb, dstb, den, x_stack, b_stack, zacc)


def _fuse_body(emb_ref, av_ref, out_ref):
    e = emb_ref[...]
    av = av_ref[...]
    logits = jnp.sum(e * av[:, None, :], axis=-1)
    m = jnp.max(logits, axis=0, keepdims=True)
    w = jnp.exp(logits - m)
    w = w / jnp.sum(w, axis=0, keepdims=True)
    out_ref[...] = e[0] * w[0][:, None] + e[1] * w[1][:, None]


def _fuse(emb, av):
    return pl.pallas_call(
        _fuse_body,
        out_shape=jax.ShapeDtypeStruct((NQF, EMB), jnp.float32),
    )(emb, av)


def _lookup_body(tab_hbm, idx_hbm, out_hbm, idxb, rowsb, sem):
    c = lax.axis_index("c")
    s = lax.axis_index("s")
    wid = s * 2 + c
    pltpu.sync_copy(idx_hbm.at[wid], idxb)

    def chunk(j, _):
        hs = []
        for g in range(LCH // IDXW):
            hs.append(pltpu.async_copy(
                tab_hbm.at[idxb.at[5 * j + g]],
                rowsb.at[pl.ds(IDXW * g, IDXW)], sem))
        for h in hs:
            h.wait()
        pltpu.sync_copy(rowsb, out_hbm.at[pl.ds(wid * TPW + j * LCH, LCH)])
        return ()

    lax.fori_loop(0, TPW // LCH, chunk, ())


def _lookup(tab, idx3d):
    mesh = plsc.VectorSubcoreMesh(core_axis_name="c", subcore_axis_name="s",
                                  num_cores=2, num_subcores=16)
    f = pl.kernel(
        _lookup_body,
        out_type=jax.ShapeDtypeStruct((TOK, EMB), jnp.float32),
        mesh=mesh,
        scratch_types=[
            pltpu.VMEM((TPW // IDXW, IDXW), jnp.int32),
            pltpu.VMEM((LCH, EMB), jnp.float32),
            pltpu.SemaphoreType.DMA,
        ],
    )
    return f(tab, idx3d)


def _blockdiag128(a):
    m = (a[:, :, None] * jnp.eye(H, dtype=a.dtype)[:, None, :]).reshape(EMB, H)
    return jnp.pad(m, ((0, 0), (0, EMB - H)))


def kernel(x1, x2, edge_index1, edge_index2, pad_ques, W1, al1, ar1, b1,
           W2, al2, ar2, b2, attnVec):
    x_stack3 = jnp.stack([x1, x2])
    w_stack = jnp.stack([W1, W2])
    ael = jnp.stack([_blockdiag128(al1), _blockdiag128(al2)])
    aer = jnp.stack([_blockdiag128(ar1), _blockdiag128(ar2)])

    feat3, el3, er3 = _prep(x_stack3, w_stack, ael, aer)
    feat = jnp.pad(feat3.reshape(P * N, EMB), ((0, NT - P * N), (0, 0)))
    el16 = jnp.pad(el3.reshape(P * N, EMB), ((0, NT - P * N), (0, 0)))
    er16 = jnp.pad(er3.reshape(P * N, EMB), ((0, NT - P * N), (0, 0)))

    pad_e = E_PAD - E
    i32 = jnp.int32
    srcb = jnp.concatenate(
        [jnp.stack([edge_index1[0], edge_index2[0] + N]),
         jnp.zeros((P, pad_e), i32)], axis=1)
    dstb = jnp.concatenate(
        [jnp.stack([edge_index1[1], edge_index2[1] + N]),
         jnp.stack([jnp.full((pad_e,), DUM, i32),
                    jnp.full((pad_e,), DUM + N, i32)])], axis=1)

    x_stack = jnp.concatenate([x1, x2])
    b_stack = jnp.stack([b1, b2])
    zacc = jnp.zeros((ZROWS, EMB), jnp.float32)

    den = _den_kernel(el16, er16, srcb, dstb, zacc)
    emb = _acc_kernel(feat, el16, er16, srcb, dstb, den, x_stack,
                      b_stack, zacc)

    ques_emb = _fuse(emb, attnVec.reshape(1, EMB))

    idx3d = pad_ques.reshape(NW, TPW // IDXW, IDXW)
    out = _lookup(ques_emb, idx3d)
    return out.reshape(B, L, EMB)
